# baseline (speedup 1.0000x reference)
import functools

import jax
import jax.numpy as jnp
from jax import lax
from jax.experimental import pallas as pl
from jax.experimental.pallas import tpu as pltpu
from jax.experimental.pallas import tpu_sc as plsc

B = 16384
DIM = 64
NCTX = 5
NC = 2
NS = 16
NW = NC * NS
BPW = B // NW
CH = 128
NCHUNK = BPW // CH
LANES = 16
NK = DIM // LANES


def _vsum(vs):
    vs = list(vs)
    while len(vs) > 1:
        vs = [a + b for a, b in zip(vs[::2], vs[1::2])] + (
            [vs[-1]] if len(vs) % 2 else [])
    return vs[0]


def _make_kernel():
    mesh = plsc.VectorSubcoreMesh(core_axis_name="c", subcore_axis_name="s")

    @functools.partial(
        pl.kernel,
        out_type=jax.ShapeDtypeStruct((B * NCTX,), jnp.float32),
        mesh=mesh,
        compiler_params=pltpu.CompilerParams(needs_layout_passes=False,
                                             use_tc_tiling_on_sc=False),
        scratch_types=[
            pltpu.VMEM((1, CH), jnp.int32),
            pltpu.VMEM((NCTX, CH), jnp.int32),
            pltpu.VMEM((CH, DIM), jnp.float32),
            pltpu.VMEM((CH * NCTX, DIM), jnp.float32),
            pltpu.VMEM((NCTX * LANES * LANES,), jnp.float32),
            pltpu.VMEM((CH * NCTX,), jnp.float32),
            pltpu.SemaphoreType.DMA,
        ],
    )
    def body(tgt_hbm, ctx_hbm, ttab_hbm, ctab_hbm, out_hbm,
             tgt_idx, ctx_idx, tgt_rows, ctx_rows, part, out_v, sem):
        wid = lax.axis_index("s") * NC + lax.axis_index("c")
        lane = lax.iota(jnp.int32, LANES)

        @pl.loop(0, NCHUNK)
        def _chunk(ch):
            rowb = wid * NCHUNK + ch
            pltpu.sync_copy(tgt_hbm.at[pl.ds(rowb * CH, CH)], tgt_idx.at[0])
            for j in range(NCTX):
                pltpu.sync_copy(
                    ctx_hbm.at[pl.ds(rowb * NCTX * CH + j * CH, CH)],
                    ctx_idx.at[j])
            cps = [pltpu.async_copy(ttab_hbm.at[tgt_idx.at[0]], tgt_rows, sem)]
            for j in range(NCTX):
                cps.append(pltpu.async_copy(
                    ctab_hbm.at[ctx_idx.at[j]],
                    ctx_rows.at[pl.ds(j * CH, CH)], sem))
            for cp in cps:
                cp.wait()

            @pl.loop(0, CH // LANES)
            def _grp(g):
                rb = g * LANES
                for r in range(LANES):
                    rr = rb + r
                    t = [tgt_rows[rr, pl.ds(k * LANES, LANES)]
                         for k in range(NK)]
                    widx = lane * LANES + r
                    for c in range(NCTX):
                        p = rr * NCTX + c
                        acc = _vsum([
                            ctx_rows[p, pl.ds(k * LANES, LANES)] * t[k]
                            for k in range(NK)])
                        plsc.store_scatter(part, [widx + c * LANES * LANES],
                                           acc)
                for c in range(NCTX):
                    s = _vsum([part[pl.ds(c * LANES * LANES + l * LANES,
                                          LANES)]
                               for l in range(LANES)])
                    oidx = lane * NCTX + (rb * NCTX + c)
                    plsc.store_scatter(out_v, [oidx], s)

            pltpu.sync_copy(out_v,
                            out_hbm.at[pl.ds(rowb * CH * NCTX, CH * NCTX)])

    return body


_sc_kernel = _make_kernel()


def kernel(target, context, target_table, context_table):
    tgt2 = target.reshape(B).astype(jnp.int32)
    ctx2 = context.reshape(B * NCTX).astype(jnp.int32)
    flat = _sc_kernel(tgt2, ctx2, target_table, context_table)
    return flat.reshape(B, NCTX)
